# baseline (speedup 1.0000x reference)
import functools

import jax
import jax.numpy as jnp
from jax import lax
from jax.experimental import pallas as pl
from jax.experimental.pallas import tpu as pltpu
from jax.experimental.pallas import tpu_sc as plsc

NUSERS = 100000
NITEMS = 100000
NHOP = 3
DIM = 64
BATCH = 4096
NNEG = 32
ROWW = NHOP * DIM
DECAY = 1e-4

NC = 2
NS = 16
NW = NC * NS
RPW = BATCH // NW
CH = 4
NCH = RPW // CH
NQ = DIM // 16

_mesh = plsc.VectorSubcoreMesh(core_axis_name="c", subcore_axis_name="s")


@functools.partial(
    pl.kernel,
    out_type=[
        jax.ShapeDtypeStruct((BATCH,), jnp.float32),
        jax.ShapeDtypeStruct((BATCH,), jnp.float32),
        jax.ShapeDtypeStruct((BATCH,), jnp.float32),
    ],
    mesh=_mesh,
    compiler_params=pltpu.CompilerParams(needs_layout_passes=False,
                                         use_tc_tiling_on_sc=False),
    scratch_types=[
        pltpu.VMEM((RPW,), jnp.int32),
        pltpu.VMEM((RPW,), jnp.int32),
        pltpu.VMEM((NCH, 128), jnp.int32),
        pltpu.VMEM((RPW + 16,), jnp.float32),
        pltpu.VMEM((RPW, ROWW), jnp.float32),
        pltpu.VMEM((RPW, ROWW), jnp.float32),
        pltpu.VMEM((2 * CH * NNEG, ROWW), jnp.float32),
        pltpu.VMEM((RPW,), jnp.float32),
        pltpu.VMEM((RPW,), jnp.float32),
        pltpu.VMEM((RPW,), jnp.float32),
        pltpu.SemaphoreType.DMA,
        pltpu.SemaphoreType.DMA,
        pltpu.SemaphoreType.DMA,
        pltpu.SemaphoreType.DMA,
    ],
)
def _sc_main(user_hbm, item_hbm, seeds_hbm, users_hbm, pos_hbm, negs_hbm,
             pos_out, neg_out, reg_out,
             users_v, pos_v, negs_v, seeds_v, u_rows, p_rows, nbuf,
             outp_v, outn_v, outr_v, sem_u, sem_p, sem_n0, sem_n1):
    wid = lax.axis_index("s") * NC + lax.axis_index("c")
    base = wid * RPW
    lane = lax.iota(jnp.int32, 16)
    third = jnp.float32(1.0 / 3.0)

    pltpu.sync_copy(users_hbm.at[pl.ds(base, RPW)], users_v)
    pltpu.sync_copy(pos_hbm.at[pl.ds(base, RPW)], pos_v)
    pltpu.sync_copy(negs_hbm.at[pl.ds(wid * NCH, NCH)], negs_v)
    pltpu.sync_copy(seeds_hbm.at[pl.ds(base, RPW)], seeds_v.at[pl.ds(0, RPW)])

    cp_u = pltpu.async_copy(user_hbm.at[users_v], u_rows, sem_u)
    cp_p = pltpu.async_copy(item_hbm.at[pos_v], p_rows, sem_p)
    pltpu.async_copy(item_hbm.at[negs_v.at[0]],
                     nbuf.at[pl.ds(0, CH * NNEG)], sem_n0)
    cp_u.wait()
    cp_p.wait()

    def hsum(vs):
        acc = vs[0]
        for v in vs[1:]:
            acc = acc + v
        return jnp.sum(acc)

    def row_body(r, carry):
        accp, accn, accr = carry
        g = r // CH
        i = r % CH

        @pl.when((i == 0) & (g + 1 < NCH) & (g % 2 == 0))
        def _():
            pltpu.async_copy(item_hbm.at[negs_v.at[g + 1]],
                             nbuf.at[pl.ds(CH * NNEG, CH * NNEG)], sem_n1)

        @pl.when((i == 0) & (g + 1 < NCH) & (g % 2 == 1))
        def _():
            pltpu.async_copy(item_hbm.at[negs_v.at[g + 1]],
                             nbuf.at[pl.ds(0, CH * NNEG)], sem_n0)

        @pl.when((i == 0) & (g % 2 == 0))
        def _():
            pltpu.make_async_copy(item_hbm.at[negs_v.at[g]],
                                  nbuf.at[pl.ds(0, CH * NNEG)], sem_n0).wait()

        @pl.when((i == 0) & (g % 2 == 1))
        def _():
            pltpu.make_async_copy(item_hbm.at[negs_v.at[g]],
                                  nbuf.at[pl.ds(CH * NNEG, CH * NNEG)],
                                  sem_n1).wait()

        s = [[u_rows[r, pl.ds(h * DIM + q * 16, 16)] for q in range(NQ)]
             for h in range(NHOP)]
        p = [[p_rows[r, pl.ds(h * DIM + q * 16, 16)] for q in range(NQ)]
             for h in range(NHOP)]
        seed = seeds_v[pl.ds(r, 16)][0]
        om = jnp.float32(1.0) - seed

        u_q = [(s[0][q] + s[1][q] + s[2][q]) * third for q in range(NQ)]
        pe_q = [(p[0][q] + p[1][q] + p[2][q]) * third for q in range(NQ)]
        pos_sc = hsum([u_q[q] * pe_q[q] for q in range(NQ)])
        reg_u = hsum([u_q[q] * u_q[q] for q in range(NQ)])
        reg_p = hsum([pe_q[q] * pe_q[q] for q in range(NQ)])
        sp = [hsum([s[h][q] * p[h][q] for q in range(NQ)]) for h in range(NHOP)]

        best_s = [jnp.float32(-jnp.inf)] * NHOP
        best_j = [jnp.int32(0)] * NHOP
        for j in range(NNEG):
            for h in range(NHOP):
                nv = [nbuf[(g % 2) * (CH * NNEG) + i * NNEG + j,
                           pl.ds(h * DIM + q * 16, 16)]
                      for q in range(NQ)]
                sn = hsum([s[h][q] * nv[q] for q in range(NQ)])
                sc = seed * sp[h] + om * sn
                upd = sc > best_s[h]
                best_s[h] = jnp.where(upd, sc, best_s[h])
                best_j[h] = jnp.where(upd, jnp.int32(j), best_j[h])

        ne_q = []
        for q in range(NQ):
            acc = None
            for h in range(NHOP):
                nb = nbuf[(g % 2) * (CH * NNEG) + i * NNEG + best_j[h],
                          pl.ds(h * DIM + q * 16, 16)]
                ch = seed * p[h][q] + om * nb
                acc = ch if acc is None else acc + ch
            ne_q.append(acc * third)
        neg_sc = hsum([u_q[q] * ne_q[q] for q in range(NQ)])
        reg_n = hsum([ne_q[q] * ne_q[q] for q in range(NQ)])

        sel = lane == (r % 16)
        accp = jnp.where(sel, pos_sc, accp)
        accn = jnp.where(sel, neg_sc, accn)
        accr = jnp.where(sel, reg_u + reg_p + reg_n, accr)

        @pl.when(r % 16 == 15)
        def _():
            outp_v[pl.ds(r - 15, 16)] = accp
            outn_v[pl.ds(r - 15, 16)] = accn
            outr_v[pl.ds(r - 15, 16)] = accr

        return (accp, accn, accr)

    zeros = jnp.zeros((16,), jnp.float32)
    lax.fori_loop(0, RPW, row_body, (zeros, zeros, zeros))

    pltpu.sync_copy(outp_v, pos_out.at[pl.ds(base, RPW)])
    pltpu.sync_copy(outn_v, neg_out.at[pl.ds(base, RPW)])
    pltpu.sync_copy(outr_v, reg_out.at[pl.ds(base, RPW)])


def _finish_body(pos_ref, neg_ref, reg_ref, out_ref):
    d = neg_ref[...] - pos_ref[...]
    mf = jnp.mean(jnp.log(1.0 + jnp.exp(d)))
    regularize = jnp.sum(reg_ref[...]) / 2.0
    emb = DECAY * regularize / BATCH
    out_ref[0] = mf + emb
    out_ref[1] = mf
    out_ref[2] = emb


_finish = pl.pallas_call(
    _finish_body,
    out_shape=jax.ShapeDtypeStruct((3,), jnp.float32),
    out_specs=pl.BlockSpec(memory_space=pltpu.SMEM),
)


def kernel(user_gcn_emb, item_gcn_emb, seed_embed, users, pos_items, neg_items):
    uf = user_gcn_emb.reshape(NUSERS, ROWW)
    itf = item_gcn_emb.reshape(NITEMS, ROWW)
    seeds = seed_embed.reshape(BATCH)
    negs2 = neg_items.reshape(BATCH * NNEG // 128, 128)
    pos_s, neg_s, reg = _sc_main(uf, itf, seeds, users, pos_items, negs2)
    out = _finish(pos_s.reshape(NW, RPW), neg_s.reshape(NW, RPW),
                  reg.reshape(NW, RPW))
    return (out[0], out[1], out[2])


# speedup vs baseline: 3.3039x; 3.3039x over previous
import functools

import jax
import jax.numpy as jnp
from jax import lax
from jax.experimental import pallas as pl
from jax.experimental.pallas import tpu as pltpu
from jax.experimental.pallas import tpu_sc as plsc

NITEMS = 100000
NHOP = 3
DIM = 64
BATCH = 4096
NNEG = 32
ROWW = NHOP * DIM
PADW = 256
DECAY = 1e-4

NC = 2
NS = 16
NW = NC * NS
RPW = BATCH // NW
CH = 2
NCH = RPW // CH
CHN = CH * NNEG
NQ = DIM // 16

_mesh = plsc.VectorSubcoreMesh(core_axis_name="c", subcore_axis_name="s")


def _padT_body(in_ref, out_ref):
    out_ref[:, 0:ROWW] = in_ref[...].T


_padT = pl.pallas_call(
    _padT_body,
    grid=(196,),
    in_specs=[pl.BlockSpec((ROWW, 512), lambda i: (0, i))],
    out_specs=pl.BlockSpec((512, PADW), lambda i: (i, 0)),
    out_shape=jax.ShapeDtypeStruct((NITEMS, PADW), jnp.float32),
)


@functools.partial(
    pl.kernel,
    out_type=[
        jax.ShapeDtypeStruct((BATCH,), jnp.float32),
        jax.ShapeDtypeStruct((BATCH,), jnp.float32),
        jax.ShapeDtypeStruct((BATCH,), jnp.float32),
    ],
    mesh=_mesh,
    compiler_params=pltpu.CompilerParams(needs_layout_passes=False,
                                         use_tc_tiling_on_sc=True),
    scratch_types=[
        pltpu.VMEM((RPW,), jnp.int32),
        pltpu.VMEM((NCH, CHN), jnp.int32),
        pltpu.VMEM((RPW + 16,), jnp.float32),
        pltpu.VMEM((RPW, PADW), jnp.float32),
        pltpu.VMEM((RPW, PADW), jnp.float32),
        pltpu.VMEM((2 * CHN, PADW), jnp.float32),
        pltpu.VMEM((RPW,), jnp.float32),
        pltpu.VMEM((RPW,), jnp.float32),
        pltpu.VMEM((RPW,), jnp.float32),
        pltpu.SemaphoreType.DMA,
        pltpu.SemaphoreType.DMA,
        pltpu.SemaphoreType.DMA,
        pltpu.SemaphoreType.DMA,
    ],
)
def _sc_main(itp_hbm, s_hbm, seeds_hbm, pos_hbm, negs_hbm,
             pos_out, neg_out, reg_out,
             pos_v, negs_v, seeds_v, u_rows, p_rows, nbuf,
             outp_v, outn_v, outr_v, sem_u, sem_p, sem_n0, sem_n1):
    wid = lax.axis_index("s") * NC + lax.axis_index("c")
    base = wid * RPW
    lane = lax.iota(jnp.int32, 16)
    third = jnp.float32(1.0 / 3.0)

    pltpu.sync_copy(pos_hbm.at[pl.ds(base, RPW)], pos_v)
    pltpu.sync_copy(negs_hbm.at[pl.ds(wid * NCH, NCH)], negs_v)
    pltpu.sync_copy(seeds_hbm.at[pl.ds(base, RPW)], seeds_v.at[pl.ds(0, RPW)])

    cp_u = pltpu.async_copy(s_hbm.at[pl.ds(base, RPW)], u_rows, sem_u)
    cp_p = pltpu.async_copy(itp_hbm.at[pos_v], p_rows, sem_p)
    pltpu.async_copy(itp_hbm.at[negs_v.at[0]],
                     nbuf.at[pl.ds(0, CHN)], sem_n0)
    cp_u.wait()
    cp_p.wait()

    def hsum(vs):
        acc = vs[0]
        for v in vs[1:]:
            acc = acc + v
        return jnp.sum(acc)

    def row_body(r, carry):
        accp, accn, accr = carry
        g = r // CH
        i = r % CH

        @pl.when((i == 0) & (g + 1 < NCH) & (g % 2 == 0))
        def _():
            pltpu.async_copy(itp_hbm.at[negs_v.at[g + 1]],
                             nbuf.at[pl.ds(CHN, CHN)], sem_n1)

        @pl.when((i == 0) & (g + 1 < NCH) & (g % 2 == 1))
        def _():
            pltpu.async_copy(itp_hbm.at[negs_v.at[g + 1]],
                             nbuf.at[pl.ds(0, CHN)], sem_n0)

        @pl.when((i == 0) & (g % 2 == 0))
        def _():
            pltpu.make_async_copy(itp_hbm.at[negs_v.at[g]],
                                  nbuf.at[pl.ds(0, CHN)], sem_n0).wait()

        @pl.when((i == 0) & (g % 2 == 1))
        def _():
            pltpu.make_async_copy(itp_hbm.at[negs_v.at[g]],
                                  nbuf.at[pl.ds(CHN, CHN)], sem_n1).wait()

        s = [[u_rows[r, pl.ds(h * DIM + q * 16, 16)] for q in range(NQ)]
             for h in range(NHOP)]
        p = [[p_rows[r, pl.ds(h * DIM + q * 16, 16)] for q in range(NQ)]
             for h in range(NHOP)]
        seed = seeds_v[pl.ds(r, 16)][0]
        om = jnp.float32(1.0) - seed

        u_q = [(s[0][q] + s[1][q] + s[2][q]) * third for q in range(NQ)]
        pe_q = [(p[0][q] + p[1][q] + p[2][q]) * third for q in range(NQ)]
        pos_sc = hsum([u_q[q] * pe_q[q] for q in range(NQ)])
        reg_u = hsum([u_q[q] * u_q[q] for q in range(NQ)])
        reg_p = hsum([pe_q[q] * pe_q[q] for q in range(NQ)])
        sp = [hsum([s[h][q] * p[h][q] for q in range(NQ)]) for h in range(NHOP)]

        nbase = (g % 2) * CHN + i * NNEG
        best_s = [jnp.float32(-jnp.inf)] * NHOP
        best_j = [jnp.int32(0)] * NHOP
        for j in range(NNEG):
            for h in range(NHOP):
                nv = [nbuf[nbase + j, pl.ds(h * DIM + q * 16, 16)]
                      for q in range(NQ)]
                sn = hsum([s[h][q] * nv[q] for q in range(NQ)])
                sc = seed * sp[h] + om * sn
                upd = sc > best_s[h]
                best_s[h] = jnp.where(upd, sc, best_s[h])
                best_j[h] = jnp.where(upd, jnp.int32(j), best_j[h])

        ne_q = []
        for q in range(NQ):
            acc = None
            for h in range(NHOP):
                nb = nbuf[nbase + best_j[h], pl.ds(h * DIM + q * 16, 16)]
                ch = seed * p[h][q] + om * nb
                acc = ch if acc is None else acc + ch
            ne_q.append(acc * third)
        neg_sc = hsum([u_q[q] * ne_q[q] for q in range(NQ)])
        reg_n = hsum([ne_q[q] * ne_q[q] for q in range(NQ)])

        sel = lane == (r % 16)
        accp = jnp.where(sel, pos_sc, accp)
        accn = jnp.where(sel, neg_sc, accn)
        accr = jnp.where(sel, reg_u + reg_p + reg_n, accr)

        @pl.when(r % 16 == 15)
        def _():
            outp_v[pl.ds(r - 15, 16)] = accp
            outn_v[pl.ds(r - 15, 16)] = accn
            outr_v[pl.ds(r - 15, 16)] = accr

        return (accp, accn, accr)

    zeros = jnp.zeros((16,), jnp.float32)
    lax.fori_loop(0, RPW, row_body, (zeros, zeros, zeros))

    pltpu.sync_copy(outp_v, pos_out.at[pl.ds(base, RPW)])
    pltpu.sync_copy(outn_v, neg_out.at[pl.ds(base, RPW)])
    pltpu.sync_copy(outr_v, reg_out.at[pl.ds(base, RPW)])


def _finish_body(pos_ref, neg_ref, reg_ref, out_ref):
    d = neg_ref[...] - pos_ref[...]
    mf = jnp.mean(jnp.log(1.0 + jnp.exp(d)))
    regularize = jnp.sum(reg_ref[...]) / 2.0
    emb = DECAY * regularize / BATCH
    out_ref[0] = mf + emb
    out_ref[1] = mf
    out_ref[2] = emb


_finish = pl.pallas_call(
    _finish_body,
    out_shape=jax.ShapeDtypeStruct((3,), jnp.float32),
    out_specs=pl.BlockSpec(memory_space=pltpu.SMEM),
)


def kernel(user_gcn_emb, item_gcn_emb, seed_embed, users, pos_items, neg_items):
    itp = _padT(item_gcn_emb.reshape(NITEMS, ROWW).T)
    s_pre = jnp.take(user_gcn_emb, users, axis=0).reshape(BATCH, ROWW)
    s_pad = jnp.pad(s_pre, ((0, 0), (0, PADW - ROWW)))
    seeds = seed_embed.reshape(BATCH)
    negs2 = neg_items.reshape(BATCH * NNEG // CHN, CHN)
    pos_s, neg_s, reg = _sc_main(itp, s_pad, seeds, pos_items, negs2)
    out = _finish(pos_s.reshape(NW, RPW), neg_s.reshape(NW, RPW),
                  reg.reshape(NW, RPW))
    return (out[0], out[1], out[2])


# speedup vs baseline: 4.1895x; 1.2681x over previous
import functools

import jax
import jax.numpy as jnp
from jax import lax
from jax.experimental import pallas as pl
from jax.experimental.pallas import tpu as pltpu
from jax.experimental.pallas import tpu_sc as plsc

NITEMS = 100000
NHOP = 3
DIM = 64
BATCH = 4096
NNEG = 32
ROWW = NHOP * DIM
PADW = 256
DECAY = 1e-4

NC = 2
NS = 16
NW = NC * NS
RPW = BATCH // NW
CH = 2
NCH = RPW // CH
CHN = CH * NNEG
NQ = DIM // 16

_mesh = plsc.VectorSubcoreMesh(core_axis_name="c", subcore_axis_name="s")


def _padT_body(in_ref, out_ref):
    out_ref[:, 0:ROWW] = in_ref[...].T


_padT = pl.pallas_call(
    _padT_body,
    grid=(49,),
    in_specs=[pl.BlockSpec((ROWW, 2048), lambda i: (0, i))],
    out_specs=pl.BlockSpec((2048, PADW), lambda i: (i, 0)),
    out_shape=jax.ShapeDtypeStruct((NITEMS, PADW), jnp.float32),
)


@functools.partial(
    pl.kernel,
    out_type=[
        jax.ShapeDtypeStruct((BATCH,), jnp.float32),
        jax.ShapeDtypeStruct((BATCH,), jnp.float32),
        jax.ShapeDtypeStruct((BATCH,), jnp.float32),
    ],
    mesh=_mesh,
    compiler_params=pltpu.CompilerParams(needs_layout_passes=False,
                                         use_tc_tiling_on_sc=True),
    scratch_types=[
        pltpu.VMEM((RPW,), jnp.int32),
        pltpu.VMEM((NCH, CHN), jnp.int32),
        pltpu.VMEM((RPW + 16,), jnp.float32),
        pltpu.VMEM((RPW, PADW), jnp.float32),
        pltpu.VMEM((RPW, PADW), jnp.float32),
        pltpu.VMEM((2 * CHN, PADW), jnp.float32),
        pltpu.VMEM((RPW,), jnp.float32),
        pltpu.VMEM((RPW,), jnp.float32),
        pltpu.VMEM((RPW,), jnp.float32),
        pltpu.SemaphoreType.DMA,
        pltpu.SemaphoreType.DMA,
        pltpu.SemaphoreType.DMA,
        pltpu.SemaphoreType.DMA,
    ],
)
def _sc_main(itp_hbm, s_hbm, seeds_hbm, pos_hbm, negs_hbm,
             pos_out, neg_out, reg_out,
             pos_v, negs_v, seeds_v, u_rows, p_rows, nbuf,
             outp_v, outn_v, outr_v, sem_u, sem_p, sem_n0, sem_n1):
    wid = lax.axis_index("s") * NC + lax.axis_index("c")
    base = wid * RPW
    lane = lax.iota(jnp.int32, 16)
    third = jnp.float32(1.0 / 3.0)

    pltpu.sync_copy(pos_hbm.at[pl.ds(base, RPW)], pos_v)
    pltpu.sync_copy(negs_hbm.at[pl.ds(wid * NCH, NCH)], negs_v)
    pltpu.sync_copy(seeds_hbm.at[pl.ds(base, RPW)], seeds_v.at[pl.ds(0, RPW)])

    cp_u = pltpu.async_copy(s_hbm.at[pl.ds(base, RPW)], u_rows, sem_u)
    cp_p = pltpu.async_copy(itp_hbm.at[pos_v], p_rows, sem_p)
    pltpu.async_copy(itp_hbm.at[negs_v.at[0]],
                     nbuf.at[pl.ds(0, CHN)], sem_n0)
    cp_u.wait()
    cp_p.wait()

    def hsum(vs):
        acc = vs[0]
        for v in vs[1:]:
            acc = acc + v
        return jnp.sum(acc)

    def row_body(r, carry):
        accp, accn, accr = carry
        g = r // CH
        i = r % CH

        @pl.when((i == 0) & (g + 1 < NCH) & (g % 2 == 0))
        def _():
            pltpu.async_copy(itp_hbm.at[negs_v.at[g + 1]],
                             nbuf.at[pl.ds(CHN, CHN)], sem_n1)

        @pl.when((i == 0) & (g + 1 < NCH) & (g % 2 == 1))
        def _():
            pltpu.async_copy(itp_hbm.at[negs_v.at[g + 1]],
                             nbuf.at[pl.ds(0, CHN)], sem_n0)

        @pl.when((i == 0) & (g % 2 == 0))
        def _():
            pltpu.make_async_copy(itp_hbm.at[negs_v.at[g]],
                                  nbuf.at[pl.ds(0, CHN)], sem_n0).wait()

        @pl.when((i == 0) & (g % 2 == 1))
        def _():
            pltpu.make_async_copy(itp_hbm.at[negs_v.at[g]],
                                  nbuf.at[pl.ds(CHN, CHN)], sem_n1).wait()

        s = [[u_rows[r, pl.ds(h * DIM + q * 16, 16)] for q in range(NQ)]
             for h in range(NHOP)]
        p = [[p_rows[r, pl.ds(h * DIM + q * 16, 16)] for q in range(NQ)]
             for h in range(NHOP)]
        seed = seeds_v[pl.ds(r, 16)][0]
        om = jnp.float32(1.0) - seed

        u_q = [(s[0][q] + s[1][q] + s[2][q]) * third for q in range(NQ)]
        pe_q = [(p[0][q] + p[1][q] + p[2][q]) * third for q in range(NQ)]
        pos_sc = hsum([u_q[q] * pe_q[q] for q in range(NQ)])
        reg_u = hsum([u_q[q] * u_q[q] for q in range(NQ)])
        reg_p = hsum([pe_q[q] * pe_q[q] for q in range(NQ)])
        sp = [hsum([s[h][q] * p[h][q] for q in range(NQ)]) for h in range(NHOP)]

        nbase = (g % 2) * CHN + i * NNEG
        best_s = [jnp.float32(-jnp.inf)] * NHOP
        best_j = [jnp.int32(0)] * NHOP
        for j in range(NNEG):
            for h in range(NHOP):
                nv = [nbuf[nbase + j, pl.ds(h * DIM + q * 16, 16)]
                      for q in range(NQ)]
                sn = hsum([s[h][q] * nv[q] for q in range(NQ)])
                sc = seed * sp[h] + om * sn
                upd = sc > best_s[h]
                best_s[h] = jnp.where(upd, sc, best_s[h])
                best_j[h] = jnp.where(upd, jnp.int32(j), best_j[h])

        ne_q = []
        for q in range(NQ):
            acc = None
            for h in range(NHOP):
                nb = nbuf[nbase + best_j[h], pl.ds(h * DIM + q * 16, 16)]
                ch = seed * p[h][q] + om * nb
                acc = ch if acc is None else acc + ch
            ne_q.append(acc * third)
        neg_sc = hsum([u_q[q] * ne_q[q] for q in range(NQ)])
        reg_n = hsum([ne_q[q] * ne_q[q] for q in range(NQ)])

        sel = lane == (r % 16)
        accp = jnp.where(sel, pos_sc, accp)
        accn = jnp.where(sel, neg_sc, accn)
        accr = jnp.where(sel, reg_u + reg_p + reg_n, accr)

        @pl.when(r % 16 == 15)
        def _():
            outp_v[pl.ds(r - 15, 16)] = accp
            outn_v[pl.ds(r - 15, 16)] = accn
            outr_v[pl.ds(r - 15, 16)] = accr

        return (accp, accn, accr)

    zeros = jnp.zeros((16,), jnp.float32)
    lax.fori_loop(0, RPW, row_body, (zeros, zeros, zeros))

    pltpu.sync_copy(outp_v, pos_out.at[pl.ds(base, RPW)])
    pltpu.sync_copy(outn_v, neg_out.at[pl.ds(base, RPW)])
    pltpu.sync_copy(outr_v, reg_out.at[pl.ds(base, RPW)])


def _finish_body(pos_ref, neg_ref, reg_ref, out_ref):
    d = neg_ref[...] - pos_ref[...]
    mf = jnp.mean(jnp.log(1.0 + jnp.exp(d)))
    regularize = jnp.sum(reg_ref[...]) / 2.0
    emb = DECAY * regularize / BATCH
    out_ref[0] = mf + emb
    out_ref[1] = mf
    out_ref[2] = emb


_finish = pl.pallas_call(
    _finish_body,
    out_shape=jax.ShapeDtypeStruct((3,), jnp.float32),
    out_specs=pl.BlockSpec(memory_space=pltpu.SMEM),
)


def kernel(user_gcn_emb, item_gcn_emb, seed_embed, users, pos_items, neg_items):
    itp = _padT(item_gcn_emb.reshape(NITEMS, ROWW).T)
    s_pre = jnp.take(user_gcn_emb, users, axis=0).reshape(BATCH, ROWW)
    s_pad = jnp.pad(s_pre, ((0, 0), (0, PADW - ROWW)))
    seeds = seed_embed.reshape(BATCH)
    negs2 = neg_items.reshape(BATCH * NNEG // CHN, CHN)
    pos_s, neg_s, reg = _sc_main(itp, s_pad, seeds, pos_items, negs2)
    out = _finish(pos_s.reshape(NW, RPW), neg_s.reshape(NW, RPW),
                  reg.reshape(NW, RPW))
    return (out[0], out[1], out[2])


# speedup vs baseline: 4.2512x; 1.0147x over previous
import functools

import jax
import jax.numpy as jnp
from jax import lax
from jax.experimental import pallas as pl
from jax.experimental.pallas import tpu as pltpu
from jax.experimental.pallas import tpu_sc as plsc

NITEMS = 100000
NHOP = 3
DIM = 64
BATCH = 4096
NNEG = 32
ROWW = NHOP * DIM
PADW = 256
DECAY = 1e-4

NC = 2
NS = 16
NW = NC * NS
RPW = BATCH // NW
CH = 2
NCH = RPW // CH
CHN = CH * NNEG
NQ = DIM // 16

_mesh = plsc.VectorSubcoreMesh(core_axis_name="c", subcore_axis_name="s")


def _padT_body(in_ref, out_ref):
    out_ref[:, 0:ROWW] = in_ref[...].T


_padT = pl.pallas_call(
    _padT_body,
    grid=(25,),
    in_specs=[pl.BlockSpec((ROWW, 4096), lambda i: (0, i))],
    out_specs=pl.BlockSpec((4096, PADW), lambda i: (i, 0)),
    out_shape=jax.ShapeDtypeStruct((NITEMS, PADW), jnp.float32),
)


@functools.partial(
    pl.kernel,
    out_type=[
        jax.ShapeDtypeStruct((BATCH,), jnp.float32),
        jax.ShapeDtypeStruct((BATCH,), jnp.float32),
        jax.ShapeDtypeStruct((BATCH,), jnp.float32),
    ],
    mesh=_mesh,
    compiler_params=pltpu.CompilerParams(needs_layout_passes=False,
                                         use_tc_tiling_on_sc=True),
    scratch_types=[
        pltpu.VMEM((RPW,), jnp.int32),
        pltpu.VMEM((NCH, CHN), jnp.int32),
        pltpu.VMEM((RPW + 16,), jnp.float32),
        pltpu.VMEM((RPW, PADW), jnp.float32),
        pltpu.VMEM((RPW, PADW), jnp.float32),
        pltpu.VMEM((2 * CHN, PADW), jnp.float32),
        pltpu.VMEM((RPW,), jnp.float32),
        pltpu.VMEM((RPW,), jnp.float32),
        pltpu.VMEM((RPW,), jnp.float32),
        pltpu.SemaphoreType.DMA,
        pltpu.SemaphoreType.DMA,
        pltpu.SemaphoreType.DMA,
        pltpu.SemaphoreType.DMA,
    ],
)
def _sc_main(itp_hbm, s_hbm, seeds_hbm, pos_hbm, negs_hbm,
             pos_out, neg_out, reg_out,
             pos_v, negs_v, seeds_v, u_rows, p_rows, nbuf,
             outp_v, outn_v, outr_v, sem_u, sem_p, sem_n0, sem_n1):
    wid = lax.axis_index("s") * NC + lax.axis_index("c")
    base = wid * RPW
    lane = lax.iota(jnp.int32, 16)
    third = jnp.float32(1.0 / 3.0)

    pltpu.sync_copy(pos_hbm.at[pl.ds(base, RPW)], pos_v)
    pltpu.sync_copy(negs_hbm.at[pl.ds(wid * NCH, NCH)], negs_v)
    pltpu.sync_copy(seeds_hbm.at[pl.ds(base, RPW)], seeds_v.at[pl.ds(0, RPW)])

    cp_u = pltpu.async_copy(s_hbm.at[pl.ds(base, RPW)], u_rows, sem_u)
    cp_p = pltpu.async_copy(itp_hbm.at[pos_v], p_rows, sem_p)
    pltpu.async_copy(itp_hbm.at[negs_v.at[0]],
                     nbuf.at[pl.ds(0, CHN)], sem_n0)
    cp_u.wait()
    cp_p.wait()

    def hsum(vs):
        acc = vs[0]
        for v in vs[1:]:
            acc = acc + v
        return jnp.sum(acc)

    def row_body(r, carry):
        accp, accn, accr = carry
        g = r // CH
        i = r % CH

        @pl.when((i == 0) & (g + 1 < NCH) & (g % 2 == 0))
        def _():
            pltpu.async_copy(itp_hbm.at[negs_v.at[g + 1]],
                             nbuf.at[pl.ds(CHN, CHN)], sem_n1)

        @pl.when((i == 0) & (g + 1 < NCH) & (g % 2 == 1))
        def _():
            pltpu.async_copy(itp_hbm.at[negs_v.at[g + 1]],
                             nbuf.at[pl.ds(0, CHN)], sem_n0)

        @pl.when((i == 0) & (g % 2 == 0))
        def _():
            pltpu.make_async_copy(itp_hbm.at[negs_v.at[g]],
                                  nbuf.at[pl.ds(0, CHN)], sem_n0).wait()

        @pl.when((i == 0) & (g % 2 == 1))
        def _():
            pltpu.make_async_copy(itp_hbm.at[negs_v.at[g]],
                                  nbuf.at[pl.ds(CHN, CHN)], sem_n1).wait()

        s = [[u_rows[r, pl.ds(h * DIM + q * 16, 16)] for q in range(NQ)]
             for h in range(NHOP)]
        p = [[p_rows[r, pl.ds(h * DIM + q * 16, 16)] for q in range(NQ)]
             for h in range(NHOP)]
        seed = seeds_v[pl.ds(r, 16)][0]
        om = jnp.float32(1.0) - seed

        u_q = [(s[0][q] + s[1][q] + s[2][q]) * third for q in range(NQ)]
        pe_q = [(p[0][q] + p[1][q] + p[2][q]) * third for q in range(NQ)]
        pos_sc = hsum([u_q[q] * pe_q[q] for q in range(NQ)])
        reg_u = hsum([u_q[q] * u_q[q] for q in range(NQ)])
        reg_p = hsum([pe_q[q] * pe_q[q] for q in range(NQ)])
        sp = [hsum([s[h][q] * p[h][q] for q in range(NQ)]) for h in range(NHOP)]

        nbase = (g % 2) * CHN + i * NNEG
        best_s = [jnp.float32(-jnp.inf)] * NHOP
        best_j = [jnp.int32(0)] * NHOP
        for j in range(NNEG):
            for h in range(NHOP):
                nv = [nbuf[nbase + j, pl.ds(h * DIM + q * 16, 16)]
                      for q in range(NQ)]
                sn = hsum([s[h][q] * nv[q] for q in range(NQ)])
                sc = seed * sp[h] + om * sn
                upd = sc > best_s[h]
                best_s[h] = jnp.where(upd, sc, best_s[h])
                best_j[h] = jnp.where(upd, jnp.int32(j), best_j[h])

        ne_q = []
        for q in range(NQ):
            acc = None
            for h in range(NHOP):
                nb = nbuf[nbase + best_j[h], pl.ds(h * DIM + q * 16, 16)]
                ch = seed * p[h][q] + om * nb
                acc = ch if acc is None else acc + ch
            ne_q.append(acc * third)
        neg_sc = hsum([u_q[q] * ne_q[q] for q in range(NQ)])
        reg_n = hsum([ne_q[q] * ne_q[q] for q in range(NQ)])

        sel = lane == (r % 16)
        accp = jnp.where(sel, pos_sc, accp)
        accn = jnp.where(sel, neg_sc, accn)
        accr = jnp.where(sel, reg_u + reg_p + reg_n, accr)

        @pl.when(r % 16 == 15)
        def _():
            outp_v[pl.ds(r - 15, 16)] = accp
            outn_v[pl.ds(r - 15, 16)] = accn
            outr_v[pl.ds(r - 15, 16)] = accr

        return (accp, accn, accr)

    zeros = jnp.zeros((16,), jnp.float32)
    lax.fori_loop(0, RPW, row_body, (zeros, zeros, zeros))

    pltpu.sync_copy(outp_v, pos_out.at[pl.ds(base, RPW)])
    pltpu.sync_copy(outn_v, neg_out.at[pl.ds(base, RPW)])
    pltpu.sync_copy(outr_v, reg_out.at[pl.ds(base, RPW)])


def _finish_body(pos_ref, neg_ref, reg_ref, out_ref):
    d = neg_ref[...] - pos_ref[...]
    mf = jnp.mean(jnp.log(1.0 + jnp.exp(d)))
    regularize = jnp.sum(reg_ref[...]) / 2.0
    emb = DECAY * regularize / BATCH
    out_ref[0] = mf + emb
    out_ref[1] = mf
    out_ref[2] = emb


_finish = pl.pallas_call(
    _finish_body,
    out_shape=jax.ShapeDtypeStruct((3,), jnp.float32),
    out_specs=pl.BlockSpec(memory_space=pltpu.SMEM),
)


def kernel(user_gcn_emb, item_gcn_emb, seed_embed, users, pos_items, neg_items):
    itp = _padT(item_gcn_emb.reshape(NITEMS, ROWW).T)
    s_pre = jnp.take(user_gcn_emb, users, axis=0).reshape(BATCH, ROWW)
    s_pad = jnp.pad(s_pre, ((0, 0), (0, PADW - ROWW)))
    seeds = seed_embed.reshape(BATCH)
    negs2 = neg_items.reshape(BATCH * NNEG // CHN, CHN)
    pos_s, neg_s, reg = _sc_main(itp, s_pad, seeds, pos_items, negs2)
    out = _finish(pos_s.reshape(NW, RPW), neg_s.reshape(NW, RPW),
                  reg.reshape(NW, RPW))
    return (out[0], out[1], out[2])


# speedup vs baseline: 4.2797x; 1.0067x over previous
import functools

import jax
import jax.numpy as jnp
from jax import lax
from jax.experimental import pallas as pl
from jax.experimental.pallas import tpu as pltpu
from jax.experimental.pallas import tpu_sc as plsc

NITEMS = 100000
NHOP = 3
DIM = 64
BATCH = 4096
NNEG = 32
ROWW = NHOP * DIM
PADW = 256
DECAY = 1e-4

NC = 2
NS = 16
NW = NC * NS
RPW = BATCH // NW
CH = 2
NCH = RPW // CH
CHN = CH * NNEG
NQ = DIM // 16

_mesh = plsc.VectorSubcoreMesh(core_axis_name="c", subcore_axis_name="s")


def _padT_body(in_ref, out_ref):
    out_ref[:, 0:ROWW] = in_ref[...].T


_padT = pl.pallas_call(
    _padT_body,
    grid=(13,),
    in_specs=[pl.BlockSpec((ROWW, 8192), lambda i: (0, i))],
    out_specs=pl.BlockSpec((8192, PADW), lambda i: (i, 0)),
    out_shape=jax.ShapeDtypeStruct((NITEMS, PADW), jnp.float32),
)


@functools.partial(
    pl.kernel,
    out_type=[
        jax.ShapeDtypeStruct((BATCH,), jnp.float32),
        jax.ShapeDtypeStruct((BATCH,), jnp.float32),
        jax.ShapeDtypeStruct((BATCH,), jnp.float32),
    ],
    mesh=_mesh,
    compiler_params=pltpu.CompilerParams(needs_layout_passes=False,
                                         use_tc_tiling_on_sc=True),
    scratch_types=[
        pltpu.VMEM((RPW,), jnp.int32),
        pltpu.VMEM((NCH, CHN), jnp.int32),
        pltpu.VMEM((RPW + 16,), jnp.float32),
        pltpu.VMEM((RPW, PADW), jnp.float32),
        pltpu.VMEM((RPW, PADW), jnp.float32),
        pltpu.VMEM((2 * CHN, PADW), jnp.float32),
        pltpu.VMEM((RPW,), jnp.float32),
        pltpu.VMEM((RPW,), jnp.float32),
        pltpu.VMEM((RPW,), jnp.float32),
        pltpu.SemaphoreType.DMA,
        pltpu.SemaphoreType.DMA,
        pltpu.SemaphoreType.DMA,
        pltpu.SemaphoreType.DMA,
    ],
)
def _sc_main(itp_hbm, s_hbm, seeds_hbm, pos_hbm, negs_hbm,
             pos_out, neg_out, reg_out,
             pos_v, negs_v, seeds_v, u_rows, p_rows, nbuf,
             outp_v, outn_v, outr_v, sem_u, sem_p, sem_n0, sem_n1):
    wid = lax.axis_index("s") * NC + lax.axis_index("c")
    base = wid * RPW
    lane = lax.iota(jnp.int32, 16)
    third = jnp.float32(1.0 / 3.0)

    pltpu.sync_copy(pos_hbm.at[pl.ds(base, RPW)], pos_v)
    pltpu.sync_copy(negs_hbm.at[pl.ds(wid * NCH, NCH)], negs_v)
    pltpu.sync_copy(seeds_hbm.at[pl.ds(base, RPW)], seeds_v.at[pl.ds(0, RPW)])

    cp_u = pltpu.async_copy(s_hbm.at[pl.ds(base, RPW)], u_rows, sem_u)
    cp_p = pltpu.async_copy(itp_hbm.at[pos_v], p_rows, sem_p)
    pltpu.async_copy(itp_hbm.at[negs_v.at[0]],
                     nbuf.at[pl.ds(0, CHN)], sem_n0)
    cp_u.wait()
    cp_p.wait()

    def hsum(vs):
        acc = vs[0]
        for v in vs[1:]:
            acc = acc + v
        return jnp.sum(acc)

    def row_body(r, carry):
        accp, accn, accr = carry
        g = r // CH
        i = r % CH

        @pl.when((i == 0) & (g + 1 < NCH) & (g % 2 == 0))
        def _():
            pltpu.async_copy(itp_hbm.at[negs_v.at[g + 1]],
                             nbuf.at[pl.ds(CHN, CHN)], sem_n1)

        @pl.when((i == 0) & (g + 1 < NCH) & (g % 2 == 1))
        def _():
            pltpu.async_copy(itp_hbm.at[negs_v.at[g + 1]],
                             nbuf.at[pl.ds(0, CHN)], sem_n0)

        @pl.when((i == 0) & (g % 2 == 0))
        def _():
            pltpu.make_async_copy(itp_hbm.at[negs_v.at[g]],
                                  nbuf.at[pl.ds(0, CHN)], sem_n0).wait()

        @pl.when((i == 0) & (g % 2 == 1))
        def _():
            pltpu.make_async_copy(itp_hbm.at[negs_v.at[g]],
                                  nbuf.at[pl.ds(CHN, CHN)], sem_n1).wait()

        s = [[u_rows[r, pl.ds(h * DIM + q * 16, 16)] for q in range(NQ)]
             for h in range(NHOP)]
        p = [[p_rows[r, pl.ds(h * DIM + q * 16, 16)] for q in range(NQ)]
             for h in range(NHOP)]
        seed = seeds_v[pl.ds(r, 16)][0]
        om = jnp.float32(1.0) - seed

        u_q = [(s[0][q] + s[1][q] + s[2][q]) * third for q in range(NQ)]
        pe_q = [(p[0][q] + p[1][q] + p[2][q]) * third for q in range(NQ)]
        pos_sc = hsum([u_q[q] * pe_q[q] for q in range(NQ)])
        reg_u = hsum([u_q[q] * u_q[q] for q in range(NQ)])
        reg_p = hsum([pe_q[q] * pe_q[q] for q in range(NQ)])
        sp = [hsum([s[h][q] * p[h][q] for q in range(NQ)]) for h in range(NHOP)]

        nbase = (g % 2) * CHN + i * NNEG
        best_s = [jnp.float32(-jnp.inf)] * NHOP
        best_j = [jnp.int32(0)] * NHOP
        for j in range(NNEG):
            for h in range(NHOP):
                nv = [nbuf[nbase + j, pl.ds(h * DIM + q * 16, 16)]
                      for q in range(NQ)]
                sn = hsum([s[h][q] * nv[q] for q in range(NQ)])
                sc = seed * sp[h] + om * sn
                upd = sc > best_s[h]
                best_s[h] = jnp.where(upd, sc, best_s[h])
                best_j[h] = jnp.where(upd, jnp.int32(j), best_j[h])

        ne_q = []
        for q in range(NQ):
            acc = None
            for h in range(NHOP):
                nb = nbuf[nbase + best_j[h], pl.ds(h * DIM + q * 16, 16)]
                ch = seed * p[h][q] + om * nb
                acc = ch if acc is None else acc + ch
            ne_q.append(acc * third)
        neg_sc = hsum([u_q[q] * ne_q[q] for q in range(NQ)])
        reg_n = hsum([ne_q[q] * ne_q[q] for q in range(NQ)])

        sel = lane == (r % 16)
        accp = jnp.where(sel, pos_sc, accp)
        accn = jnp.where(sel, neg_sc, accn)
        accr = jnp.where(sel, reg_u + reg_p + reg_n, accr)

        @pl.when(r % 16 == 15)
        def _():
            outp_v[pl.ds(r - 15, 16)] = accp
            outn_v[pl.ds(r - 15, 16)] = accn
            outr_v[pl.ds(r - 15, 16)] = accr

        return (accp, accn, accr)

    zeros = jnp.zeros((16,), jnp.float32)
    lax.fori_loop(0, RPW, row_body, (zeros, zeros, zeros))

    pltpu.sync_copy(outp_v, pos_out.at[pl.ds(base, RPW)])
    pltpu.sync_copy(outn_v, neg_out.at[pl.ds(base, RPW)])
    pltpu.sync_copy(outr_v, reg_out.at[pl.ds(base, RPW)])


def _finish_body(pos_ref, neg_ref, reg_ref, out_ref):
    d = neg_ref[...] - pos_ref[...]
    mf = jnp.mean(jnp.log(1.0 + jnp.exp(d)))
    regularize = jnp.sum(reg_ref[...]) / 2.0
    emb = DECAY * regularize / BATCH
    out_ref[0] = mf + emb
    out_ref[1] = mf
    out_ref[2] = emb


_finish = pl.pallas_call(
    _finish_body,
    out_shape=jax.ShapeDtypeStruct((3,), jnp.float32),
    out_specs=pl.BlockSpec(memory_space=pltpu.SMEM),
)


def kernel(user_gcn_emb, item_gcn_emb, seed_embed, users, pos_items, neg_items):
    itp = _padT(item_gcn_emb.reshape(NITEMS, ROWW).T)
    s_pre = jnp.take(user_gcn_emb, users, axis=0).reshape(BATCH, ROWW)
    s_pad = jnp.pad(s_pre, ((0, 0), (0, PADW - ROWW)))
    seeds = seed_embed.reshape(BATCH)
    negs2 = neg_items.reshape(BATCH * NNEG // CHN, CHN)
    pos_s, neg_s, reg = _sc_main(itp, s_pad, seeds, pos_items, negs2)
    out = _finish(pos_s.reshape(NW, RPW), neg_s.reshape(NW, RPW),
                  reg.reshape(NW, RPW))
    return (out[0], out[1], out[2])
